# baseline (speedup 1.0000x reference)
import dataclasses
import functools

import jax
import jax.numpy as jnp
from jax import lax
from jax.experimental import pallas as pl
from jax.experimental.pallas import tpu as pltpu
from jax.experimental.pallas import tpu_sc as plsc

H = 128
NC = 2
NS = 16
L = 16
NW = NC * NS
KB = 128

NP = 10240
RPT = NP // NS
TR = NP // H
SHIFT = 14
MASK = (1 << SHIFT) - 1


def _round_up(a, b):
    return (a + b - 1) // b * b


def _sc_layer(h, packed3, a_src, a_dst, nb):
    mesh = plsc.VectorSubcoreMesh(core_axis_name="c", subcore_axis_name="s")
    cp = pltpu.CompilerParams()
    if "needs_layout_passes" in pltpu.CompilerParams.__dataclass_fields__:
        cp = dataclasses.replace(cp, needs_layout_passes=False)

    @functools.partial(
        pl.kernel,
        compiler_params=cp,
        out_type=[
            jax.ShapeDtypeStruct((NC, NP, H), jnp.float32),
            jax.ShapeDtypeStruct((NC, NP), jnp.float32),
        ],
        mesh=mesh,
        scratch_types=[
            pltpu.VMEM((nb, KB), jnp.int32),
            pltpu.VMEM((KB,), jnp.float32),
            pltpu.VMEM((2 * TR, 128), jnp.float32),
            pltpu.VMEM((KB, 128), jnp.float32),
            pltpu.VMEM((KB,), jnp.int32),
            pltpu.VMEM((KB,), jnp.int32),
            pltpu.SemaphoreType.DMA,
            pltpu.VMEM_SHARED((NP, H), jnp.float32),
            pltpu.VMEM_SHARED((NP,), jnp.float32),
        ],
    )
    def k(h_hbm, pk_hbm, asrc_hbm, adst_hbm, outp_hbm, denp_hbm,
          pk2d, eexp_b, tab, rowbuf, src_b, dst_b, sem, out_sp, den_sp):
        c = lax.axis_index("c")
        s = lax.axis_index("s")
        wid = c * NS + s
        row0 = s * RPT

        pltpu.sync_copy(pk_hbm.at[wid], pk2d)

        @pl.loop(0, KB)
        def _(r):
            for j in range(H // L):
                rowbuf[r, pl.ds(j * L, L)] = jnp.zeros((L,), jnp.float32)

        @pl.loop(0, RPT, step=KB)
        def _(r):
            pltpu.sync_copy(rowbuf, out_sp.at[pl.ds(row0 + r, KB)])
            pltpu.sync_copy(rowbuf.at[0], den_sp.at[pl.ds(row0 + r, KB)])

        pltpu.sync_copy(asrc_hbm, tab.at[pl.ds(0, TR)])
        pltpu.sync_copy(adst_hbm, tab.at[pl.ds(TR, TR)])

        plsc.subcore_barrier()

        @pl.loop(0, nb)
        def _(b):
            @pl.loop(0, KB, step=L)
            def _(i):
                pk = pk2d[b, pl.ds(i, L)]
                src_b[pl.ds(i, L)] = pk & MASK
                dst_b[pl.ds(i, L)] = lax.shift_right_logical(pk, SHIFT)

            cp_rows = pltpu.async_copy(h_hbm.at[src_b], rowbuf, sem)

            @pl.loop(0, KB, step=L)
            def _(i):
                si = src_b[pl.ds(i, L)]
                di = dst_b[pl.ds(i, L)]
                av = plsc.load_gather(
                    tab, [lax.shift_right_logical(si, 7), si & 127])
                dv = plsc.load_gather(
                    tab, [TR + lax.shift_right_logical(di, 7), di & 127])
                lv = av + dv
                lv = jnp.maximum(lv, 0.2 * lv)
                eexp_b[pl.ds(i, L)] = jnp.exp(lv)

            pltpu.sync_copy(eexp_b, den_sp.at[dst_b], add=True)
            cp_rows.wait()

            @pl.loop(0, KB, step=L)
            def _(i):
                ev = eexp_b[pl.ds(i, L)]

                for t in range(L):
                    al = lax.gather(
                        ev, jnp.full((L, 1), t, jnp.int32),
                        lax.GatherDimensionNumbers(
                            offset_dims=(), collapsed_slice_dims=(0,),
                            start_index_map=(0,)),
                        slice_sizes=(1,),
                        mode=lax.GatherScatterMode.PROMISE_IN_BOUNDS)
                    for j in range(H // L):
                        rowbuf[i + t, pl.ds(j * L, L)] = (
                            rowbuf[i + t, pl.ds(j * L, L)] * al)

            pltpu.sync_copy(rowbuf, out_sp.at[dst_b], add=True)

        plsc.subcore_barrier()

        pltpu.sync_copy(out_sp.at[pl.ds(row0, RPT)],
                        outp_hbm.at[c, pl.ds(row0, RPT)])
        pltpu.sync_copy(den_sp.at[pl.ds(row0, RPT)],
                        denp_hbm.at[c, pl.ds(row0, RPT)])

    return k(h, packed3, a_src, a_dst)


_BLK = 1024

_EXT_OUT = [
    jax.ShapeDtypeStruct((NP, H), jnp.float32),
    jax.ShapeDtypeStruct((TR, 128), jnp.float32),
    jax.ShapeDtypeStruct((TR, 128), jnp.float32),
]
_EXT_OUT_SPECS = [
    pl.BlockSpec((_BLK, H), lambda i: (i, 0)),
    pl.BlockSpec((_BLK // H, 128), lambda i: (i, 0)),
    pl.BlockSpec((_BLK // H, 128), lambda i: (i, 0)),
]


def _split_cols(hx, h_ref, as_ref, ad_ref):
    h_ref[...] = hx[:, :H]
    as_ref[...] = hx[:, H].reshape(_BLK // H, 128)
    ad_ref[...] = hx[:, H + 1].reshape(_BLK // H, 128)


def _tc_first(xp, wext):
    def body(x_ref, w_ref, h_ref, as_ref, ad_ref):
        hx = jnp.dot(x_ref[...], w_ref[...],
                     preferred_element_type=jnp.float32,
                     precision=lax.Precision.HIGHEST)
        _split_cols(hx, h_ref, as_ref, ad_ref)

    return pl.pallas_call(
        body,
        grid=(NP // _BLK,),
        in_specs=[pl.BlockSpec((_BLK, H), lambda i: (i, 0)),
                  pl.BlockSpec((H, 2 * H), lambda i: (0, 0))],
        out_specs=_EXT_OUT_SPECS,
        out_shape=_EXT_OUT,
    )(xp, wext)


def _norm_relu(p_ref, d_ref, b_ref):
    i = pl.program_id(0)
    p = p_ref[0] + p_ref[1]
    den = d_ref[0, pl.ds(i * _BLK, _BLK)] + d_ref[1, pl.ds(i * _BLK, _BLK)]
    den = den + 1e-16
    return jnp.maximum(p / den[:, None] + b_ref[...], 0.0)


_NORM_IN_SPECS = [
    pl.BlockSpec((NC, _BLK, H), lambda i: (0, i, 0)),
    pl.BlockSpec((NC, NP), lambda i: (0, 0)),
    pl.BlockSpec((H,), lambda i: (0,)),
]


def _tc_mid(outp, denp, bias, wext):
    def body(p_ref, d_ref, b_ref, w_ref, h_ref, as_ref, ad_ref):
        g = _norm_relu(p_ref, d_ref, b_ref)
        hx = jnp.dot(g, w_ref[...],
                     preferred_element_type=jnp.float32,
                     precision=lax.Precision.HIGHEST)
        _split_cols(hx, h_ref, as_ref, ad_ref)

    return pl.pallas_call(
        body,
        grid=(NP // _BLK,),
        in_specs=_NORM_IN_SPECS + [pl.BlockSpec((H, 2 * H), lambda i: (0, 0))],
        out_specs=_EXT_OUT_SPECS,
        out_shape=_EXT_OUT,
    )(outp, denp, bias, wext)


def _tc_final(outp, denp, bias, w3, b3):
    def body(p_ref, d_ref, b_ref, w_ref, b3_ref, o_ref):
        g = _norm_relu(p_ref, d_ref, b_ref)
        o_ref[...] = jnp.dot(g, w_ref[...],
                             preferred_element_type=jnp.float32,
                             precision=lax.Precision.HIGHEST) + b3_ref[...]

    return pl.pallas_call(
        body,
        grid=(NP // _BLK,),
        in_specs=_NORM_IN_SPECS + [pl.BlockSpec((H, H), lambda i: (0, 0)),
                                   pl.BlockSpec((H,), lambda i: (0,))],
        out_specs=pl.BlockSpec((_BLK, H), lambda i: (i, 0)),
        out_shape=jax.ShapeDtypeStruct((NP, H), jnp.float32),
    )(outp, denp, bias, w3, b3)


def kernel(x, edge_index, W1, att_src1, att_dst1, b1,
           W2, att_src2, att_dst2, b2, W3, b3):
    n = x.shape[0]
    e = edge_index.shape[1]
    e_tot = e + n
    nb = _round_up(e_tot, NW * KB) // (NW * KB)
    e_pad = NW * KB * nb

    loops = jnp.arange(n, dtype=jnp.int32)
    pad = e_pad - e_tot
    src = jnp.concatenate([edge_index[0], loops,
                           jnp.zeros((pad,), jnp.int32)])
    dst = jnp.concatenate([edge_index[1], loops,
                           jnp.full((pad,), n, jnp.int32)])
    packed3 = (src | (dst << SHIFT)).reshape(NW, nb, KB)

    xp = jnp.zeros((NP, H), jnp.float32).at[:n].set(x)

    def ext_weights(W, att_s, att_d):
        cols = jnp.zeros((H, H), jnp.float32)
        cols = cols.at[:, 0].set(W @ att_s).at[:, 1].set(W @ att_d)
        return jnp.concatenate([W, cols], axis=1)

    h1, as1, ad1 = _tc_first(xp, ext_weights(W1, att_src1, att_dst1))
    outp1, denp1 = _sc_layer(h1, packed3, as1, ad1, nb)

    h2, as2, ad2 = _tc_mid(outp1, denp1, b1,
                           ext_weights(W2, att_src2, att_dst2))
    outp2, denp2 = _sc_layer(h2, packed3, as2, ad2, nb)

    out = _tc_final(outp2, denp2, b2, W3, b3)
    return out[:n]


# speedup vs baseline: 1.2995x; 1.2995x over previous
import dataclasses
import functools

import jax
import jax.numpy as jnp
from jax import lax
from jax.experimental import pallas as pl
from jax.experimental.pallas import tpu as pltpu
from jax.experimental.pallas import tpu_sc as plsc

H = 128
NC = 2
NS = 16
L = 16
NW = NC * NS
KB = 128
KBH = 64

NP = 10240
RPT = NP // NS
TR = NP // H
SHIFT = 14
MASK = (1 << SHIFT) - 1


def _round_up(a, b):
    return (a + b - 1) // b * b


def _sc_layer(h, packed3, a_src, a_dst, nh):
    mesh = plsc.VectorSubcoreMesh(core_axis_name="c", subcore_axis_name="s")
    cp = pltpu.CompilerParams()
    if "needs_layout_passes" in pltpu.CompilerParams.__dataclass_fields__:
        cp = dataclasses.replace(cp, needs_layout_passes=False)

    @functools.partial(
        pl.kernel,
        compiler_params=cp,
        out_type=[
            jax.ShapeDtypeStruct((NC, NP, H), jnp.float32),
            jax.ShapeDtypeStruct((NC, NP), jnp.float32),
        ],
        mesh=mesh,
        scratch_types=[
            pltpu.VMEM((KBH,), jnp.int32),
            pltpu.VMEM((KBH,), jnp.int32),
            pltpu.SemaphoreType.DMA,
            pltpu.SemaphoreType.DMA,
            pltpu.VMEM((2 * TR, 128), jnp.float32),
            pltpu.VMEM((KBH, 128), jnp.float32),
            pltpu.VMEM((KBH, 128), jnp.float32),
            pltpu.VMEM((KBH,), jnp.float32),
            pltpu.VMEM((KBH,), jnp.float32),
            pltpu.VMEM((KBH,), jnp.int32),
            pltpu.VMEM((KBH,), jnp.int32),
            pltpu.VMEM((KBH,), jnp.int32),
            pltpu.VMEM((KBH,), jnp.int32),
            pltpu.VMEM((KBH,), jnp.int32),
            pltpu.VMEM((KBH,), jnp.int32),
            pltpu.SemaphoreType.DMA,
            pltpu.SemaphoreType.DMA,
            pltpu.SemaphoreType.DMA,
            pltpu.SemaphoreType.DMA,
            pltpu.VMEM_SHARED((NP, H), jnp.float32),
            pltpu.VMEM_SHARED((NP,), jnp.float32),
        ],
    )
    def k(h_hbm, pk_hbm, asrc_hbm, adst_hbm, outp_hbm, denp_hbm,
          pkb0, pkb1, p0, p1, tab, rb0, rb1, ee0, ee1,
          sb0, sb1, db0, db1, sd0, sd1,
          g0, g1, s0, s1, out_sp, den_sp):
        c = lax.axis_index("c")
        s = lax.axis_index("s")
        wid = c * NS + s
        row0 = s * RPT

        @pl.loop(0, KBH)
        def _(r):
            for j in range(H // L):
                rb0[r, pl.ds(j * L, L)] = jnp.zeros((L,), jnp.float32)

        @pl.loop(0, RPT, step=KBH)
        def _(r):
            pltpu.sync_copy(rb0, out_sp.at[pl.ds(row0 + r, KBH)])

        @pl.loop(0, RPT, step=128)
        def _(r):
            pltpu.sync_copy(rb0.at[0], den_sp.at[pl.ds(row0 + r, 128)])

        pltpu.sync_copy(asrc_hbm, tab.at[pl.ds(0, TR)])
        pltpu.sync_copy(adst_hbm, tab.at[pl.ds(TR, TR)])

        plsc.subcore_barrier()

        bufs = [(pkb0, p0, rb0, ee0, sb0, db0, sd0, g0, s0),
                (pkb1, p1, rb1, ee1, sb1, db1, sd1, g1, s1)]

        def prep(hh, pkb, ps, rb, ee, sb, db, sd, gs, ss, drain):
            if drain:
                pltpu.make_async_copy(pk_hbm.at[wid, hh], pkb, ps).wait()

            @pl.loop(0, KBH, step=L)
            def _(i):
                pk = pkb[pl.ds(i, L)]
                sb[pl.ds(i, L)] = pk & MASK
                db[pl.ds(i, L)] = lax.shift_right_logical(pk, SHIFT)

            nxt = jnp.minimum(hh + 2, nh - 1)
            pltpu.async_copy(pk_hbm.at[wid, nxt], pkb, ps)

            @pl.loop(0, KBH, step=L)
            def _(i):
                si = sb[pl.ds(i, L)]
                di = db[pl.ds(i, L)]
                av = plsc.load_gather(
                    tab, [lax.shift_right_logical(si, 7), si & 127])
                dv = plsc.load_gather(
                    tab, [TR + lax.shift_right_logical(di, 7), di & 127])
                lv = av + dv
                lv = jnp.maximum(lv, 0.2 * lv)
                ee[pl.ds(i, L)] = jnp.exp(lv)

            pltpu.sync_copy(ee, den_sp.at[db], add=True)
            if drain:
                pltpu.make_async_copy(rb, out_sp.at[sd], ss).wait()
            pltpu.async_copy(h_hbm.at[sb], rb, gs)

        def consume(pkb, ps, rb, ee, sb, db, sd, gs, ss):
            pltpu.make_async_copy(h_hbm.at[sb], rb, gs).wait()

            @pl.loop(0, KBH, step=L)
            def _(i):
                ev = ee[pl.ds(i, L)]

                @pl.loop(0, L)
                def _(t):
                    al = lax.gather(
                        ev, jnp.full((L, 1), t, jnp.int32),
                        lax.GatherDimensionNumbers(
                            offset_dims=(), collapsed_slice_dims=(0,),
                            start_index_map=(0,)),
                        slice_sizes=(1,),
                        mode=lax.GatherScatterMode.PROMISE_IN_BOUNDS)
                    for j in range(H // L):
                        rb[i + t, pl.ds(j * L, L)] = (
                            rb[i + t, pl.ds(j * L, L)] * al)

            @pl.loop(0, KBH, step=L)
            def _(i):
                sd[pl.ds(i, L)] = db[pl.ds(i, L)]

            pltpu.async_copy(rb, out_sp.at[sd], ss, add=True)

        pltpu.sync_copy(pk_hbm.at[wid, 0], pkb0)
        pltpu.sync_copy(pk_hbm.at[wid, 1], pkb1)
        prep(0, *bufs[0], drain=False)
        prep(1, *bufs[1], drain=False)

        @pl.loop(0, nh // 2 - 1)
        def _(g):
            consume(*bufs[0])
            prep(2 * g + 2, *bufs[0], drain=True)
            consume(*bufs[1])
            prep(2 * g + 3, *bufs[1], drain=True)

        consume(*bufs[0])
        consume(*bufs[1])
        pltpu.make_async_copy(rb0, out_sp.at[sd0], s0).wait()
        pltpu.make_async_copy(rb1, out_sp.at[sd1], s1).wait()
        pltpu.make_async_copy(pk_hbm.at[wid, nh - 1], pkb0, p0).wait()
        pltpu.make_async_copy(pk_hbm.at[wid, nh - 1], pkb1, p1).wait()

        plsc.subcore_barrier()

        pltpu.sync_copy(out_sp.at[pl.ds(row0, RPT)],
                        outp_hbm.at[c, pl.ds(row0, RPT)])
        pltpu.sync_copy(den_sp.at[pl.ds(row0, RPT)],
                        denp_hbm.at[c, pl.ds(row0, RPT)])

    return k(h, packed3, a_src, a_dst)


_BLK = 1024

_EXT_OUT = [
    jax.ShapeDtypeStruct((NP, H), jnp.float32),
    jax.ShapeDtypeStruct((TR, 128), jnp.float32),
    jax.ShapeDtypeStruct((TR, 128), jnp.float32),
]
_EXT_OUT_SPECS = [
    pl.BlockSpec((_BLK, H), lambda i: (i, 0)),
    pl.BlockSpec((_BLK // H, 128), lambda i: (i, 0)),
    pl.BlockSpec((_BLK // H, 128), lambda i: (i, 0)),
]


def _split_cols(hx, h_ref, as_ref, ad_ref):
    h_ref[...] = hx[:, :H]
    as_ref[...] = hx[:, H].reshape(_BLK // H, 128)
    ad_ref[...] = hx[:, H + 1].reshape(_BLK // H, 128)


def _tc_first(xp, wext):
    def body(x_ref, w_ref, h_ref, as_ref, ad_ref):
        hx = jnp.dot(x_ref[...], w_ref[...],
                     preferred_element_type=jnp.float32,
                     precision=lax.Precision.HIGHEST)
        _split_cols(hx, h_ref, as_ref, ad_ref)

    return pl.pallas_call(
        body,
        grid=(NP // _BLK,),
        in_specs=[pl.BlockSpec((_BLK, H), lambda i: (i, 0)),
                  pl.BlockSpec((H, 2 * H), lambda i: (0, 0))],
        out_specs=_EXT_OUT_SPECS,
        out_shape=_EXT_OUT,
    )(xp, wext)


def _norm_relu(p_ref, d_ref, b_ref):
    i = pl.program_id(0)
    p = p_ref[0] + p_ref[1]
    den = d_ref[0, pl.ds(i * _BLK, _BLK)] + d_ref[1, pl.ds(i * _BLK, _BLK)]
    den = den + 1e-16
    return jnp.maximum(p / den[:, None] + b_ref[...], 0.0)


_NORM_IN_SPECS = [
    pl.BlockSpec((NC, _BLK, H), lambda i: (0, i, 0)),
    pl.BlockSpec((NC, NP), lambda i: (0, 0)),
    pl.BlockSpec((H,), lambda i: (0,)),
]


def _tc_mid(outp, denp, bias, wext):
    def body(p_ref, d_ref, b_ref, w_ref, h_ref, as_ref, ad_ref):
        g = _norm_relu(p_ref, d_ref, b_ref)
        hx = jnp.dot(g, w_ref[...],
                     preferred_element_type=jnp.float32,
                     precision=lax.Precision.HIGHEST)
        _split_cols(hx, h_ref, as_ref, ad_ref)

    return pl.pallas_call(
        body,
        grid=(NP // _BLK,),
        in_specs=_NORM_IN_SPECS + [pl.BlockSpec((H, 2 * H), lambda i: (0, 0))],
        out_specs=_EXT_OUT_SPECS,
        out_shape=_EXT_OUT,
    )(outp, denp, bias, wext)


def _tc_final(outp, denp, bias, w3, b3):
    def body(p_ref, d_ref, b_ref, w_ref, b3_ref, o_ref):
        g = _norm_relu(p_ref, d_ref, b_ref)
        o_ref[...] = jnp.dot(g, w_ref[...],
                             preferred_element_type=jnp.float32,
                             precision=lax.Precision.HIGHEST) + b3_ref[...]

    return pl.pallas_call(
        body,
        grid=(NP // _BLK,),
        in_specs=_NORM_IN_SPECS + [pl.BlockSpec((H, H), lambda i: (0, 0)),
                                   pl.BlockSpec((H,), lambda i: (0,))],
        out_specs=pl.BlockSpec((_BLK, H), lambda i: (i, 0)),
        out_shape=jax.ShapeDtypeStruct((NP, H), jnp.float32),
    )(outp, denp, bias, w3, b3)


def kernel(x, edge_index, W1, att_src1, att_dst1, b1,
           W2, att_src2, att_dst2, b2, W3, b3):
    n = x.shape[0]
    e = edge_index.shape[1]
    e_tot = e + n
    nb = _round_up(e_tot, NW * KB) // (NW * KB)
    e_pad = NW * KB * nb

    loops = jnp.arange(n, dtype=jnp.int32)
    pad = e_pad - e_tot
    src = jnp.concatenate([edge_index[0], loops,
                           jnp.zeros((pad,), jnp.int32)])
    dst = jnp.concatenate([edge_index[1], loops,
                           jnp.full((pad,), n, jnp.int32)])
    nh = 2 * nb
    packed3 = (src | (dst << SHIFT)).reshape(NW, nh, KBH)

    xp = jnp.zeros((NP, H), jnp.float32).at[:n].set(x)

    def ext_weights(W, att_s, att_d):
        cols = jnp.zeros((H, H), jnp.float32)
        cols = cols.at[:, 0].set(W @ att_s).at[:, 1].set(W @ att_d)
        return jnp.concatenate([W, cols], axis=1)

    h1, as1, ad1 = _tc_first(xp, ext_weights(W1, att_src1, att_dst1))
    outp1, denp1 = _sc_layer(h1, packed3, as1, ad1, nh)

    h2, as2, ad2 = _tc_mid(outp1, denp1, b1,
                           ext_weights(W2, att_src2, att_dst2))
    outp2, denp2 = _sc_layer(h2, packed3, as2, ad2, nh)

    out = _tc_final(outp2, denp2, b2, W3, b3)
    return out[:n]


# speedup vs baseline: 1.3277x; 1.0217x over previous
import dataclasses
import functools

import jax
import jax.numpy as jnp
from jax import lax
from jax.experimental import pallas as pl
from jax.experimental.pallas import tpu as pltpu
from jax.experimental.pallas import tpu_sc as plsc

H = 128
NC = 2
NS = 16
L = 16
NW = NC * NS
KB = 128
KBH = 64

NP = 10240
RPT = NP // NS
TR = NP // H
SHIFT = 14
MASK = (1 << SHIFT) - 1


def _round_up(a, b):
    return (a + b - 1) // b * b


def _sc_layer(h, packed3, a_src, a_dst, nh):
    mesh = plsc.VectorSubcoreMesh(core_axis_name="c", subcore_axis_name="s")
    cp = pltpu.CompilerParams()
    if "needs_layout_passes" in pltpu.CompilerParams.__dataclass_fields__:
        cp = dataclasses.replace(cp, needs_layout_passes=False)

    @functools.partial(
        pl.kernel,
        compiler_params=cp,
        out_type=[
            jax.ShapeDtypeStruct((NC, NP, H), jnp.float32),
            jax.ShapeDtypeStruct((NC, NP), jnp.float32),
        ],
        mesh=mesh,
        scratch_types=[
            pltpu.VMEM((KBH,), jnp.int32),
            pltpu.VMEM((KBH,), jnp.int32),
            pltpu.SemaphoreType.DMA,
            pltpu.SemaphoreType.DMA,
            pltpu.VMEM((2 * TR, 128), jnp.float32),
            pltpu.VMEM((KBH, 128), jnp.float32),
            pltpu.VMEM((KBH, 128), jnp.float32),
            pltpu.VMEM((KBH,), jnp.float32),
            pltpu.VMEM((KBH,), jnp.float32),
            pltpu.VMEM((KBH,), jnp.int32),
            pltpu.VMEM((KBH,), jnp.int32),
            pltpu.VMEM((KBH,), jnp.int32),
            pltpu.VMEM((KBH,), jnp.int32),
            pltpu.VMEM((KBH,), jnp.int32),
            pltpu.VMEM((KBH,), jnp.int32),
            pltpu.SemaphoreType.DMA,
            pltpu.SemaphoreType.DMA,
            pltpu.SemaphoreType.DMA,
            pltpu.SemaphoreType.DMA,
            pltpu.SemaphoreType.DMA,
            pltpu.SemaphoreType.DMA,
            pltpu.VMEM_SHARED((NP, H), jnp.float32),
            pltpu.VMEM_SHARED((NP,), jnp.float32),
        ],
    )
    def k(h_hbm, pk_hbm, asrc_hbm, adst_hbm, outp_hbm, denp_hbm,
          pkb0, pkb1, p0, p1, tab, rb0, rb1, ee0, ee1,
          sb0, sb1, db0, db1, sd0, sd1,
          g0, g1, s0, s1, d0, d1, out_sp, den_sp):
        c = lax.axis_index("c")
        s = lax.axis_index("s")
        wid = c * NS + s
        row0 = s * RPT

        @pl.loop(0, KBH)
        def _(r):
            for j in range(H // L):
                rb0[r, pl.ds(j * L, L)] = jnp.zeros((L,), jnp.float32)

        @pl.loop(0, RPT, step=KBH)
        def _(r):
            pltpu.sync_copy(rb0, out_sp.at[pl.ds(row0 + r, KBH)])

        @pl.loop(0, RPT, step=128)
        def _(r):
            pltpu.sync_copy(rb0.at[0], den_sp.at[pl.ds(row0 + r, 128)])

        pltpu.sync_copy(asrc_hbm, tab.at[pl.ds(0, TR)])
        pltpu.sync_copy(adst_hbm, tab.at[pl.ds(TR, TR)])

        plsc.subcore_barrier()

        bufs = [(pkb0, p0, rb0, ee0, sb0, db0, sd0, g0, s0, d0),
                (pkb1, p1, rb1, ee1, sb1, db1, sd1, g1, s1, d1)]

        def prep(hh, pkb, ps, rb, ee, sb, db, sd, gs, ss, ds_, drain):
            if drain:
                pltpu.make_async_copy(pk_hbm.at[wid, hh], pkb, ps).wait()
                pltpu.make_async_copy(ee, den_sp.at[db], ds_).wait()

            @pl.loop(0, KBH, step=L)
            def _(i):
                pk = pkb[pl.ds(i, L)]
                sb[pl.ds(i, L)] = pk & MASK
                db[pl.ds(i, L)] = lax.shift_right_logical(pk, SHIFT)

            nxt = jnp.minimum(hh + 2, nh - 1)
            pltpu.async_copy(pk_hbm.at[wid, nxt], pkb, ps)

            @pl.loop(0, KBH, step=L)
            def _(i):
                si = sb[pl.ds(i, L)]
                di = db[pl.ds(i, L)]
                av = plsc.load_gather(
                    tab, [lax.shift_right_logical(si, 7), si & 127])
                dv = plsc.load_gather(
                    tab, [TR + lax.shift_right_logical(di, 7), di & 127])
                lv = av + dv
                lv = jnp.maximum(lv, 0.2 * lv)
                ee[pl.ds(i, L)] = jnp.exp(lv)

            pltpu.async_copy(ee, den_sp.at[db], ds_, add=True)
            if drain:
                pltpu.make_async_copy(rb, out_sp.at[sd], ss).wait()
            pltpu.async_copy(h_hbm.at[sb], rb, gs)

        def consume(pkb, ps, rb, ee, sb, db, sd, gs, ss, ds_):
            pltpu.make_async_copy(h_hbm.at[sb], rb, gs).wait()

            @pl.loop(0, KBH, step=L)
            def _(i):
                ev = ee[pl.ds(i, L)]

                @pl.loop(0, L)
                def _(t):
                    al = lax.gather(
                        ev, jnp.full((L, 1), t, jnp.int32),
                        lax.GatherDimensionNumbers(
                            offset_dims=(), collapsed_slice_dims=(0,),
                            start_index_map=(0,)),
                        slice_sizes=(1,),
                        mode=lax.GatherScatterMode.PROMISE_IN_BOUNDS)
                    for j in range(H // L):
                        rb[i + t, pl.ds(j * L, L)] = (
                            rb[i + t, pl.ds(j * L, L)] * al)

            @pl.loop(0, KBH, step=L)
            def _(i):
                sd[pl.ds(i, L)] = db[pl.ds(i, L)]

            pltpu.async_copy(rb, out_sp.at[sd], ss, add=True)

        pltpu.sync_copy(pk_hbm.at[wid, 0], pkb0)
        pltpu.sync_copy(pk_hbm.at[wid, 1], pkb1)
        prep(0, *bufs[0], drain=False)
        prep(1, *bufs[1], drain=False)

        @pl.loop(0, nh // 2 - 1)
        def _(g):
            consume(*bufs[0])
            prep(2 * g + 2, *bufs[0], drain=True)
            consume(*bufs[1])
            prep(2 * g + 3, *bufs[1], drain=True)

        consume(*bufs[0])
        consume(*bufs[1])
        pltpu.make_async_copy(rb0, out_sp.at[sd0], s0).wait()
        pltpu.make_async_copy(rb1, out_sp.at[sd1], s1).wait()
        pltpu.make_async_copy(pk_hbm.at[wid, nh - 1], pkb0, p0).wait()
        pltpu.make_async_copy(pk_hbm.at[wid, nh - 1], pkb1, p1).wait()
        pltpu.make_async_copy(ee0, den_sp.at[db0], d0).wait()
        pltpu.make_async_copy(ee1, den_sp.at[db1], d1).wait()

        plsc.subcore_barrier()

        pltpu.sync_copy(out_sp.at[pl.ds(row0, RPT)],
                        outp_hbm.at[c, pl.ds(row0, RPT)])
        pltpu.sync_copy(den_sp.at[pl.ds(row0, RPT)],
                        denp_hbm.at[c, pl.ds(row0, RPT)])

    return k(h, packed3, a_src, a_dst)


_BLK = 1024

_EXT_OUT = [
    jax.ShapeDtypeStruct((NP, H), jnp.float32),
    jax.ShapeDtypeStruct((TR, 128), jnp.float32),
    jax.ShapeDtypeStruct((TR, 128), jnp.float32),
]
_EXT_OUT_SPECS = [
    pl.BlockSpec((_BLK, H), lambda i: (i, 0)),
    pl.BlockSpec((_BLK // H, 128), lambda i: (i, 0)),
    pl.BlockSpec((_BLK // H, 128), lambda i: (i, 0)),
]


def _split_cols(hx, h_ref, as_ref, ad_ref):
    h_ref[...] = hx[:, :H]
    as_ref[...] = hx[:, H].reshape(_BLK // H, 128)
    ad_ref[...] = hx[:, H + 1].reshape(_BLK // H, 128)


def _tc_first(xp, wext):
    def body(x_ref, w_ref, h_ref, as_ref, ad_ref):
        hx = jnp.dot(x_ref[...], w_ref[...],
                     preferred_element_type=jnp.float32,
                     precision=lax.Precision.HIGHEST)
        _split_cols(hx, h_ref, as_ref, ad_ref)

    return pl.pallas_call(
        body,
        grid=(NP // _BLK,),
        in_specs=[pl.BlockSpec((_BLK, H), lambda i: (i, 0)),
                  pl.BlockSpec((H, 2 * H), lambda i: (0, 0))],
        out_specs=_EXT_OUT_SPECS,
        out_shape=_EXT_OUT,
    )(xp, wext)


def _norm_relu(p_ref, d_ref, b_ref):
    i = pl.program_id(0)
    p = p_ref[0] + p_ref[1]
    den = d_ref[0, pl.ds(i * _BLK, _BLK)] + d_ref[1, pl.ds(i * _BLK, _BLK)]
    den = den + 1e-16
    return jnp.maximum(p / den[:, None] + b_ref[...], 0.0)


_NORM_IN_SPECS = [
    pl.BlockSpec((NC, _BLK, H), lambda i: (0, i, 0)),
    pl.BlockSpec((NC, NP), lambda i: (0, 0)),
    pl.BlockSpec((H,), lambda i: (0,)),
]


def _tc_mid(outp, denp, bias, wext):
    def body(p_ref, d_ref, b_ref, w_ref, h_ref, as_ref, ad_ref):
        g = _norm_relu(p_ref, d_ref, b_ref)
        hx = jnp.dot(g, w_ref[...],
                     preferred_element_type=jnp.float32,
                     precision=lax.Precision.HIGHEST)
        _split_cols(hx, h_ref, as_ref, ad_ref)

    return pl.pallas_call(
        body,
        grid=(NP // _BLK,),
        in_specs=_NORM_IN_SPECS + [pl.BlockSpec((H, 2 * H), lambda i: (0, 0))],
        out_specs=_EXT_OUT_SPECS,
        out_shape=_EXT_OUT,
    )(outp, denp, bias, wext)


def _tc_final(outp, denp, bias, w3, b3):
    def body(p_ref, d_ref, b_ref, w_ref, b3_ref, o_ref):
        g = _norm_relu(p_ref, d_ref, b_ref)
        o_ref[...] = jnp.dot(g, w_ref[...],
                             preferred_element_type=jnp.float32,
                             precision=lax.Precision.HIGHEST) + b3_ref[...]

    return pl.pallas_call(
        body,
        grid=(NP // _BLK,),
        in_specs=_NORM_IN_SPECS + [pl.BlockSpec((H, H), lambda i: (0, 0)),
                                   pl.BlockSpec((H,), lambda i: (0,))],
        out_specs=pl.BlockSpec((_BLK, H), lambda i: (i, 0)),
        out_shape=jax.ShapeDtypeStruct((NP, H), jnp.float32),
    )(outp, denp, bias, w3, b3)


def kernel(x, edge_index, W1, att_src1, att_dst1, b1,
           W2, att_src2, att_dst2, b2, W3, b3):
    n = x.shape[0]
    e = edge_index.shape[1]
    e_tot = e + n
    nb = _round_up(e_tot, NW * KB) // (NW * KB)
    e_pad = NW * KB * nb

    loops = jnp.arange(n, dtype=jnp.int32)
    pad = e_pad - e_tot
    src = jnp.concatenate([edge_index[0], loops,
                           jnp.zeros((pad,), jnp.int32)])
    dst = jnp.concatenate([edge_index[1], loops,
                           jnp.full((pad,), n, jnp.int32)])
    nh = 2 * nb
    packed3 = (src | (dst << SHIFT)).reshape(nh, NW, KBH).transpose(1, 0, 2)

    xp = jnp.zeros((NP, H), jnp.float32).at[:n].set(x)

    def ext_weights(W, att_s, att_d):
        cols = jnp.zeros((H, H), jnp.float32)
        cols = cols.at[:, 0].set(W @ att_s).at[:, 1].set(W @ att_d)
        return jnp.concatenate([W, cols], axis=1)

    h1, as1, ad1 = _tc_first(xp, ext_weights(W1, att_src1, att_dst1))
    outp1, denp1 = _sc_layer(h1, packed3, as1, ad1, nh)

    h2, as2, ad2 = _tc_mid(outp1, denp1, b1,
                           ext_weights(W2, att_src2, att_dst2))
    outp2, denp2 = _sc_layer(h2, packed3, as2, ad2, nh)

    out = _tc_final(outp2, denp2, b2, W3, b3)
    return out[:n]


# speedup vs baseline: 1.3287x; 1.0007x over previous
import dataclasses
import functools

import jax
import jax.numpy as jnp
from jax import lax
from jax.experimental import pallas as pl
from jax.experimental.pallas import tpu as pltpu
from jax.experimental.pallas import tpu_sc as plsc

H = 128
NC = 2
NS = 16
L = 16
NW = NC * NS
KB = 128
KBH = 64

NP = 10240
RPT = NP // NS
TR = NP // H
SHIFT = 14
MASK = (1 << SHIFT) - 1


def _round_up(a, b):
    return (a + b - 1) // b * b


def _sc_layer(h, packed3, a_src, a_dst, nh):
    mesh = plsc.VectorSubcoreMesh(core_axis_name="c", subcore_axis_name="s")
    cp = pltpu.CompilerParams()
    if "needs_layout_passes" in pltpu.CompilerParams.__dataclass_fields__:
        cp = dataclasses.replace(cp, needs_layout_passes=False)

    @functools.partial(
        pl.kernel,
        compiler_params=cp,
        out_type=[
            jax.ShapeDtypeStruct((NC, NP, H), jnp.float32),
            jax.ShapeDtypeStruct((NC, NP), jnp.float32),
        ],
        mesh=mesh,
        scratch_types=[
            pltpu.VMEM((KBH,), jnp.int32),
            pltpu.VMEM((KBH,), jnp.int32),
            pltpu.SemaphoreType.DMA,
            pltpu.SemaphoreType.DMA,
            pltpu.VMEM((2 * TR, 128), jnp.float32),
            pltpu.VMEM((KBH, 128), jnp.float32),
            pltpu.VMEM((KBH, 128), jnp.float32),
            pltpu.VMEM((KBH,), jnp.float32),
            pltpu.VMEM((KBH,), jnp.float32),
            pltpu.VMEM((KBH,), jnp.int32),
            pltpu.VMEM((KBH,), jnp.int32),
            pltpu.VMEM((KBH,), jnp.int32),
            pltpu.VMEM((KBH,), jnp.int32),
            pltpu.VMEM((KBH,), jnp.int32),
            pltpu.VMEM((KBH,), jnp.int32),
            pltpu.SemaphoreType.DMA,
            pltpu.SemaphoreType.DMA,
            pltpu.SemaphoreType.DMA,
            pltpu.SemaphoreType.DMA,
            pltpu.SemaphoreType.DMA,
            pltpu.SemaphoreType.DMA,
            pltpu.VMEM_SHARED((NP, H), jnp.float32),
            pltpu.VMEM_SHARED((NP,), jnp.float32),
        ],
    )
    def k(h_hbm, pk_hbm, asrc_hbm, adst_hbm, outp_hbm, denp_hbm,
          pkb0, pkb1, p0, p1, tab, rb0, rb1, ee0, ee1,
          sb0, sb1, db0, db1, sd0, sd1,
          g0, g1, s0, s1, d0, d1, out_sp, den_sp):
        c = lax.axis_index("c")
        s = lax.axis_index("s")
        wid = c * NS + s
        row0 = s * RPT

        @pl.loop(0, KBH)
        def _(r):
            for j in range(H // L):
                rb0[r, pl.ds(j * L, L)] = jnp.zeros((L,), jnp.float32)

        @pl.loop(0, RPT, step=KBH)
        def _(r):
            pltpu.sync_copy(rb0, out_sp.at[pl.ds(row0 + r, KBH)])

        @pl.loop(0, RPT, step=128)
        def _(r):
            pltpu.sync_copy(rb0.at[0], den_sp.at[pl.ds(row0 + r, 128)])

        pltpu.sync_copy(asrc_hbm, tab.at[pl.ds(0, TR)])
        pltpu.sync_copy(adst_hbm, tab.at[pl.ds(TR, TR)])

        plsc.subcore_barrier()

        bufs = [(pkb0, p0, rb0, ee0, sb0, db0, sd0, g0, s0, d0),
                (pkb1, p1, rb1, ee1, sb1, db1, sd1, g1, s1, d1)]

        def prep(hh, pkb, ps, rb, ee, sb, db, sd, gs, ss, ds_, drain):
            if drain:
                pltpu.make_async_copy(pk_hbm.at[wid, hh], pkb, ps).wait()
                pltpu.make_async_copy(ee, den_sp.at[db], ds_).wait()

            @pl.loop(0, KBH, step=L)
            def _(i):
                pk = pkb[pl.ds(i, L)]
                sb[pl.ds(i, L)] = pk & MASK
                db[pl.ds(i, L)] = lax.shift_right_logical(pk, SHIFT)

            nxt = jnp.minimum(hh + 2, nh - 1)
            pltpu.async_copy(pk_hbm.at[wid, nxt], pkb, ps)

            @pl.loop(0, KBH, step=L)
            def _(i):
                si = sb[pl.ds(i, L)]
                di = db[pl.ds(i, L)]
                av = plsc.load_gather(
                    tab, [lax.shift_right_logical(si, 7), si & 127])
                dv = plsc.load_gather(
                    tab, [TR + lax.shift_right_logical(di, 7), di & 127])
                lv = av + dv
                lv = jnp.maximum(lv, 0.2 * lv)
                ee[pl.ds(i, L)] = jnp.exp(lv)

            pltpu.async_copy(ee, den_sp.at[db], ds_, add=True)
            if drain:
                pltpu.make_async_copy(rb, out_sp.at[sd], ss).wait()
            pltpu.async_copy(h_hbm.at[sb], rb, gs)

        def consume(pkb, ps, rb, ee, sb, db, sd, gs, ss, ds_):
            pltpu.make_async_copy(h_hbm.at[sb], rb, gs).wait()

            @plsc.parallel_loop(0, KBH, 1, unroll=4)
            def _(e):
                base = (e // L) * L
                ev = ee[pl.ds(base, L)]
                al = lax.gather(
                    ev, jnp.full((L, 1), e - base, jnp.int32),
                    lax.GatherDimensionNumbers(
                        offset_dims=(), collapsed_slice_dims=(0,),
                        start_index_map=(0,)),
                    slice_sizes=(1,),
                    mode=lax.GatherScatterMode.PROMISE_IN_BOUNDS)
                for j in range(H // L):
                    rb[e, pl.ds(j * L, L)] = rb[e, pl.ds(j * L, L)] * al

            @pl.loop(0, KBH, step=L)
            def _(i):
                sd[pl.ds(i, L)] = db[pl.ds(i, L)]

            pltpu.async_copy(rb, out_sp.at[sd], ss, add=True)

        pltpu.sync_copy(pk_hbm.at[wid, 0], pkb0)
        pltpu.sync_copy(pk_hbm.at[wid, 1], pkb1)
        prep(0, *bufs[0], drain=False)
        prep(1, *bufs[1], drain=False)

        @pl.loop(0, nh // 2 - 1)
        def _(g):
            consume(*bufs[0])
            prep(2 * g + 2, *bufs[0], drain=True)
            consume(*bufs[1])
            prep(2 * g + 3, *bufs[1], drain=True)

        consume(*bufs[0])
        consume(*bufs[1])
        pltpu.make_async_copy(rb0, out_sp.at[sd0], s0).wait()
        pltpu.make_async_copy(rb1, out_sp.at[sd1], s1).wait()
        pltpu.make_async_copy(pk_hbm.at[wid, nh - 1], pkb0, p0).wait()
        pltpu.make_async_copy(pk_hbm.at[wid, nh - 1], pkb1, p1).wait()
        pltpu.make_async_copy(ee0, den_sp.at[db0], d0).wait()
        pltpu.make_async_copy(ee1, den_sp.at[db1], d1).wait()

        plsc.subcore_barrier()

        pltpu.sync_copy(out_sp.at[pl.ds(row0, RPT)],
                        outp_hbm.at[c, pl.ds(row0, RPT)])
        pltpu.sync_copy(den_sp.at[pl.ds(row0, RPT)],
                        denp_hbm.at[c, pl.ds(row0, RPT)])

    return k(h, packed3, a_src, a_dst)


_BLK = 1024

_EXT_OUT = [
    jax.ShapeDtypeStruct((NP, H), jnp.float32),
    jax.ShapeDtypeStruct((TR, 128), jnp.float32),
    jax.ShapeDtypeStruct((TR, 128), jnp.float32),
]
_EXT_OUT_SPECS = [
    pl.BlockSpec((_BLK, H), lambda i: (i, 0)),
    pl.BlockSpec((_BLK // H, 128), lambda i: (i, 0)),
    pl.BlockSpec((_BLK // H, 128), lambda i: (i, 0)),
]


def _split_cols(hx, h_ref, as_ref, ad_ref):
    h_ref[...] = hx[:, :H]
    as_ref[...] = hx[:, H].reshape(_BLK // H, 128)
    ad_ref[...] = hx[:, H + 1].reshape(_BLK // H, 128)


def _tc_first(xp, wext):
    def body(x_ref, w_ref, h_ref, as_ref, ad_ref):
        hx = jnp.dot(x_ref[...], w_ref[...],
                     preferred_element_type=jnp.float32,
                     precision=lax.Precision.HIGHEST)
        _split_cols(hx, h_ref, as_ref, ad_ref)

    return pl.pallas_call(
        body,
        grid=(NP // _BLK,),
        in_specs=[pl.BlockSpec((_BLK, H), lambda i: (i, 0)),
                  pl.BlockSpec((H, 2 * H), lambda i: (0, 0))],
        out_specs=_EXT_OUT_SPECS,
        out_shape=_EXT_OUT,
    )(xp, wext)


def _norm_relu(p_ref, d_ref, b_ref):
    i = pl.program_id(0)
    p = p_ref[0] + p_ref[1]
    den = d_ref[0, pl.ds(i * _BLK, _BLK)] + d_ref[1, pl.ds(i * _BLK, _BLK)]
    den = den + 1e-16
    return jnp.maximum(p / den[:, None] + b_ref[...], 0.0)


_NORM_IN_SPECS = [
    pl.BlockSpec((NC, _BLK, H), lambda i: (0, i, 0)),
    pl.BlockSpec((NC, NP), lambda i: (0, 0)),
    pl.BlockSpec((H,), lambda i: (0,)),
]


def _tc_mid(outp, denp, bias, wext):
    def body(p_ref, d_ref, b_ref, w_ref, h_ref, as_ref, ad_ref):
        g = _norm_relu(p_ref, d_ref, b_ref)
        hx = jnp.dot(g, w_ref[...],
                     preferred_element_type=jnp.float32,
                     precision=lax.Precision.HIGHEST)
        _split_cols(hx, h_ref, as_ref, ad_ref)

    return pl.pallas_call(
        body,
        grid=(NP // _BLK,),
        in_specs=_NORM_IN_SPECS + [pl.BlockSpec((H, 2 * H), lambda i: (0, 0))],
        out_specs=_EXT_OUT_SPECS,
        out_shape=_EXT_OUT,
    )(outp, denp, bias, wext)


def _tc_final(outp, denp, bias, w3, b3):
    def body(p_ref, d_ref, b_ref, w_ref, b3_ref, o_ref):
        g = _norm_relu(p_ref, d_ref, b_ref)
        o_ref[...] = jnp.dot(g, w_ref[...],
                             preferred_element_type=jnp.float32,
                             precision=lax.Precision.HIGHEST) + b3_ref[...]

    return pl.pallas_call(
        body,
        grid=(NP // _BLK,),
        in_specs=_NORM_IN_SPECS + [pl.BlockSpec((H, H), lambda i: (0, 0)),
                                   pl.BlockSpec((H,), lambda i: (0,))],
        out_specs=pl.BlockSpec((_BLK, H), lambda i: (i, 0)),
        out_shape=jax.ShapeDtypeStruct((NP, H), jnp.float32),
    )(outp, denp, bias, w3, b3)


def kernel(x, edge_index, W1, att_src1, att_dst1, b1,
           W2, att_src2, att_dst2, b2, W3, b3):
    n = x.shape[0]
    e = edge_index.shape[1]
    e_tot = e + n
    nb = _round_up(e_tot, NW * KB) // (NW * KB)
    e_pad = NW * KB * nb

    loops = jnp.arange(n, dtype=jnp.int32)
    pad = e_pad - e_tot
    src = jnp.concatenate([edge_index[0], loops,
                           jnp.zeros((pad,), jnp.int32)])
    dst = jnp.concatenate([edge_index[1], loops,
                           jnp.full((pad,), n, jnp.int32)])
    nh = 2 * nb
    packed3 = (src | (dst << SHIFT)).reshape(nh, NW, KBH).transpose(1, 0, 2)

    xp = jnp.zeros((NP, H), jnp.float32).at[:n].set(x)

    def ext_weights(W, att_s, att_d):
        cols = jnp.zeros((H, H), jnp.float32)
        cols = cols.at[:, 0].set(W @ att_s).at[:, 1].set(W @ att_d)
        return jnp.concatenate([W, cols], axis=1)

    h1, as1, ad1 = _tc_first(xp, ext_weights(W1, att_src1, att_dst1))
    outp1, denp1 = _sc_layer(h1, packed3, as1, ad1, nh)

    h2, as2, ad2 = _tc_mid(outp1, denp1, b1,
                           ext_weights(W2, att_src2, att_dst2))
    outp2, denp2 = _sc_layer(h2, packed3, as2, ad2, nh)

    out = _tc_final(outp2, denp2, b2, W3, b3)
    return out[:n]
